# baseline (speedup 1.0000x reference)
import jax
import jax.numpy as jnp
from jax.experimental import pallas as pl

B = 8
N = 2048
K = 16
NF = 512


def _bn_scale(g):
    return g / jnp.sqrt(1.0 + 1e-5)


def _head_kernel(pooled_ref, w2a_ref, p2a_ref, w2b_ref, p2b_ref, out_ref):
    z = jnp.dot(pooled_ref[...], w2a_ref[...], preferred_element_type=jnp.float32)
    z = jax.nn.relu(z * p2a_ref[0:1, :] + p2a_ref[1:2, :])
    z = jnp.dot(z, w2b_ref[...], preferred_element_type=jnp.float32)
    z = jax.nn.relu(z * p2b_ref[0:1, :] + p2b_ref[1:2, :])
    out_ref[...] = z


def kernel(x, mask, w1a, b1a, g1a, be1a, w1b, b1b, g1b, be1b, w1c, b1c, g1c, be1c, gs1, gn1, gb1, gs2, gn2, gb2, w2a, b2a, g2a, be2a, w2b, b2b, g2b, be2b):
    pts = jnp.transpose(x, (0, 2, 1))
    sq = jnp.sum(pts * pts, axis=-1)
    d = sq[:, :, None] + sq[:, None, :] - 2.0 * jnp.einsum('bnd,bmd->bnm', pts, pts)
    d = jnp.where(mask[:, None, :] == 1, d, jnp.inf)
    _, idx = jax.lax.top_k(-d, K)
    gl = (idx + jnp.arange(B, dtype=idx.dtype)[:, None, None] * N).reshape(-1)

    xf = pts.reshape(-1, 3)
    neigh = xf[gl].reshape(B, N, K, 3)
    mean = jnp.mean(neigh, axis=2, keepdims=True)
    nc = neigh - mean
    cov = jnp.einsum('bnki,bnkj->bnij', nc, nc).reshape(B, N, 9)
    h = jnp.concatenate([x, jnp.transpose(cov, (0, 2, 1))], axis=1)

    def bnrelu(t, g, b):
        s = _bn_scale(g)
        return jax.nn.relu(t * s[None, :, None] + b[None, :, None])

    def conv(t, W, b):
        return jnp.einsum('oc,bcn->bon', W, t) + b[None, :, None]

    h = bnrelu(conv(h, w1a, b1a), g1a, be1a)
    h = bnrelu(conv(h, w1b, b1b), g1b, be1b)
    h = bnrelu(conv(h, w1c, b1c), g1c, be1c)

    def graph_layer(t, Ws, Wn, b):
        Bc, C, Nc = t.shape
        xt = jnp.transpose(t, (0, 2, 1))
        xf2 = xt.reshape(-1, C)
        ng = xf2[gl].reshape(Bc, Nc, K, C)
        agg = jnp.max(ng, axis=2)
        out = jax.nn.relu(xt @ Ws.T + agg @ Wn.T + b[None, None, :])
        return jnp.transpose(out, (0, 2, 1))

    h = graph_layer(h, gs1, gn1, gb1)
    h = graph_layer(h, gs2, gn2, gb2)

    mask_flat = mask.reshape(-1)
    batch_index = jnp.repeat(jnp.arange(B, dtype=mask_flat.dtype), N)
    scatter_idx = (batch_index + 1) * mask_flat
    feat = jnp.transpose(h, (0, 2, 1)).reshape(-1, 2 * NF)
    pooled = jax.ops.segment_max(feat, scatter_idx, num_segments=B + 1)[1:]

    p2a = jnp.stack([_bn_scale(g2a), be2a + b2a * 0.0], axis=0)
    p2a = jnp.stack([_bn_scale(g2a), b2a * _bn_scale(g2a) + be2a], axis=0)
    p2b = jnp.stack([_bn_scale(g2b), b2b * _bn_scale(g2b) + be2b], axis=0)

    z = pl.pallas_call(
        _head_kernel,
        out_shape=jax.ShapeDtypeStruct((B, NF), jnp.float32),
    )(pooled, w2a.T, p2a, w2b.T, p2b)
    return z


# speedup vs baseline: 8.1285x; 8.1285x over previous
import functools

import jax
import jax.numpy as jnp
from jax import lax
from jax.experimental import pallas as pl
from jax.experimental.pallas import tpu as pltpu
from jax.experimental.pallas import tpu_sc as plsc

B = 8
N = 2048
K = 16
NF = 512
BIG = 1e30
NEG = -1e37

_NC = 2
_NS = 16
NW = _NC * _NS
RPW = (B * N) // NW
RT = 256

_mesh = plsc.VectorSubcoreMesh(
    core_axis_name="c", subcore_axis_name="s", num_cores=_NC, num_subcores=_NS)


def _scale(g):
    return g / jnp.sqrt(1.0 + 1e-5)


def _t1_body(ptsA_ref, x_ref, m_ref, spen_ref, gmin_ref):
    xb = x_ref[0]
    m = m_ref[0].astype(jnp.float32)
    pen = (1.0 - m) * BIG
    x0 = xb[0:1]
    x1 = xb[1:2]
    x2 = xb[2:3]
    spen = x0 * x0 + x1 * x1 + x2 * x2 + pen
    spen_ref[0] = spen
    kb = jnp.concatenate([xb, spen, jnp.zeros((4, N), jnp.float32)], axis=0)
    a = ptsA_ref[0]
    d = lax.dot_general(a, kb, (((1,), (0,)), ((), ())),
                        preferred_element_type=jnp.float32)
    gm = d[:, 0:128]
    for i in range(1, 16):
        gm = jnp.minimum(gm, d[:, i * 128:(i + 1) * 128])
    gmin_ref[0] = gm


def _t1(ptsA, x, mask3):
    return pl.pallas_call(
        _t1_body,
        grid=(B, N // RT),
        in_specs=[
            pl.BlockSpec((1, RT, 8), lambda b, r: (b, r, 0)),
            pl.BlockSpec((1, 3, N), lambda b, r: (b, 0, 0)),
            pl.BlockSpec((1, 1, N), lambda b, r: (b, 0, 0)),
        ],
        out_specs=[
            pl.BlockSpec((1, 1, N), lambda b, r: (b, 0, 0)),
            pl.BlockSpec((1, RT, 128), lambda b, r: (b, r, 0)),
        ],
        out_shape=[
            jax.ShapeDtypeStruct((B, 1, N), jnp.float32),
            jax.ShapeDtypeStruct((B, N, 128), jnp.float32),
        ],
    )(ptsA, x, mask3)


def _vgather(vec, idx):
    dnums = lax.GatherDimensionNumbers(
        offset_dims=(), collapsed_slice_dims=(0,), start_index_map=(0,))
    return lax.gather(vec, idx[:, None], dnums, slice_sizes=(1,),
                      mode=lax.GatherScatterMode.PROMISE_IN_BOUNDS)


def _skv(k, v):
    sk, sv = lax.sort((k, v), dimension=0, num_keys=1)
    return sk, sv


def _merge16(ak, av, bk, bv):
    rk = lax.rev(bk, (0,))
    rv = lax.rev(bv, (0,))
    sel = ak <= rk
    mk = jnp.where(sel, ak, rk)
    mv = jnp.where(sel, av, rv)
    return _skv(mk, mv)


def _sc1_body(xh, spen, gminf, glo, hino,
              xt, yt, zt, st, gbuf, glbuf, hbuf):
    wid = lax.axis_index("c") * _NS + lax.axis_index("s")
    b = wid // 4
    q = wid % 4
    base = b * N

    pltpu.sync_copy(xh.at[pl.ds((b * 3 + 0) * N, N)], xt)
    pltpu.sync_copy(xh.at[pl.ds((b * 3 + 1) * N, N)], yt)
    pltpu.sync_copy(xh.at[pl.ds((b * 3 + 2) * N, N)], zt)
    pltpu.sync_copy(spen.at[pl.ds(b * N, N)], st)

    ji = lax.iota(jnp.int32, 16)
    j15 = jnp.full((16,), 15, jnp.int32)

    def chunk_body(c, carry):
        pltpu.sync_copy(
            gminf.at[pl.ds((base + q * RPW + c * 64) * 128, 64 * 128)], gbuf)

        def row_body(i2, carry2):
            row = c * 64 + i2
            n = q * RPW + row
            pairs = []
            for j in range(8):
                gj = gbuf[pl.ds(i2 * 128 + j * 16, 16)]
                kj, vj = _skv(gj, ji + (16 * j))
                pairs.append((kj, vj))
            while len(pairs) > 1:
                nxt = []
                for j in range(0, len(pairs), 2):
                    ak, av = pairs[j]
                    bk, bv = pairs[j + 1]
                    nxt.append(_merge16(ak, av, bk, bv))
                pairs = nxt
            gk, gv = pairs[0]

            nv = jnp.full((16,), n, jnp.int32)
            xr = plsc.load_gather(xt, [nv])
            yr = plsc.load_gather(yt, [nv])
            zr = plsc.load_gather(zt, [nv])

            def bstep(j, carry3):
                rk0, rv0 = carry3
                jv = jnp.full((16,), j, jnp.int32)
                gkj = _vgather(gk, jv)
                gvj = _vgather(gv, jv)
                r15 = _vgather(rk0, j15)
                go = jnp.max((r15 > gkj).astype(jnp.int32)) > 0

                def do(_):
                    cols = gvj + ji * 128
                    xc = plsc.load_gather(xt, [cols])
                    yc = plsc.load_gather(yt, [cols])
                    zc = plsc.load_gather(zt, [cols])
                    sc = plsc.load_gather(st, [cols])
                    d = sc - 2.0 * (xc * xr + yc * yr + zc * zr)
                    ck, cv = _skv(d, cols)
                    return _merge16(rk0, rv0, ck, cv)

                return lax.cond(go, do, lambda _: (rk0, rv0), None)

            rk = jnp.full((16,), 3.0e38, jnp.float32)
            rv = jnp.zeros((16,), jnp.int32)
            rk, rv = lax.fori_loop(0, 16, bstep, (rk, rv))

            glbuf[pl.ds(row * K, K)] = rv + base

            xn = plsc.load_gather(xt, [rv])
            yn = plsc.load_gather(yt, [rv])
            zn = plsc.load_gather(zt, [rv])
            sx = jnp.sum(xn)
            sy = jnp.sum(yn)
            sz = jnp.sum(zn)
            cxx = jnp.sum(xn * xn) - sx * sx * (1.0 / K)
            cxy = jnp.sum(xn * yn) - sx * sy * (1.0 / K)
            cxz = jnp.sum(xn * zn) - sx * sz * (1.0 / K)
            cyy = jnp.sum(yn * yn) - sy * sy * (1.0 / K)
            cyz = jnp.sum(yn * zn) - sy * sz * (1.0 / K)
            czz = jnp.sum(zn * zn) - sz * sz * (1.0 / K)
            hrow = jnp.zeros((16,), jnp.float32)
            hrow = jnp.where(ji == 0, xr, hrow)
            hrow = jnp.where(ji == 1, yr, hrow)
            hrow = jnp.where(ji == 2, zr, hrow)
            for lane, val in ((3, cxx), (4, cxy), (5, cxz), (6, cxy),
                              (7, cyy), (8, cyz), (9, cxz), (10, cyz),
                              (11, czz)):
                hrow = jnp.where(ji == lane, jnp.full((16,), val), hrow)
            hbuf[pl.ds(row * 16, 16)] = hrow
            return carry2

        return lax.fori_loop(0, 64, row_body, carry)

    lax.fori_loop(0, RPW // 64, chunk_body, 0)

    pltpu.sync_copy(glbuf, glo.at[pl.ds(wid * RPW * K, RPW * K)])
    pltpu.sync_copy(hbuf, hino.at[pl.ds(wid * RPW * 16, RPW * 16)])


def _sc1(x, spen2, gminf):
    fn = pl.kernel(
        _sc1_body,
        compiler_params=pltpu.CompilerParams(needs_layout_passes=False),
        out_type=(
            jax.ShapeDtypeStruct((B * N * K,), jnp.int32),
            jax.ShapeDtypeStruct((B * N * 16,), jnp.float32),
        ),
        mesh=_mesh,
        scratch_types=[
            pltpu.VMEM((N,), jnp.float32),
            pltpu.VMEM((N,), jnp.float32),
            pltpu.VMEM((N,), jnp.float32),
            pltpu.VMEM((N,), jnp.float32),
            pltpu.VMEM((64 * 128,), jnp.float32),
            pltpu.VMEM((RPW * K,), jnp.int32),
            pltpu.VMEM((RPW * 16,), jnp.float32),
        ],
    )
    return fn(x, spen2, gminf)


def _make_gathermax(C):
    CH = 8

    def body(feat, glf, outf, idxb, rowsb, outb, sem):
        wid = lax.axis_index("c") * _NS + lax.axis_index("s")
        pltpu.sync_copy(glf.at[pl.ds(wid * RPW * K, RPW * K)], idxb)

        def chunk_body(ch, carry):
            cp = pltpu.async_copy(
                feat.at[idxb.at[pl.ds(ch * CH * K, CH * K)]], rowsb, sem)
            cp.wait()
            for p in range(CH):
                for cc in range(C // 16):
                    acc = rowsb[p * K + 0, pl.ds(cc * 16, 16)]
                    for k in range(1, K):
                        acc = jnp.maximum(
                            acc, rowsb[p * K + k, pl.ds(cc * 16, 16)])
                    outb[pl.ds((ch * CH + p) * C + cc * 16, 16)] = acc
            return carry

        lax.fori_loop(0, RPW // CH, chunk_body, 0)
        pltpu.sync_copy(outb, outf.at[pl.ds(wid * RPW * C, RPW * C)])

    fn = pl.kernel(
        body,
        compiler_params=pltpu.CompilerParams(needs_layout_passes=False),
        out_type=jax.ShapeDtypeStruct((B * N * C,), jnp.float32),
        mesh=_mesh,
        scratch_types=[
            pltpu.VMEM((RPW * K,), jnp.int32),
            pltpu.VMEM((CH * K, C), jnp.float32),
            pltpu.VMEM((RPW * C,), jnp.float32),
            pltpu.SemaphoreType.DMA,
        ],
    )
    return fn


def _t2_body(h_ref, wa_ref, ba_ref, wb_ref, bb_ref, wc_ref, bc_ref, o_ref):
    h = jnp.dot(h_ref[...], wa_ref[...], preferred_element_type=jnp.float32)
    h = jax.nn.relu(h + ba_ref[...])
    h = jnp.dot(h, wb_ref[...], preferred_element_type=jnp.float32)
    h = jax.nn.relu(h + bb_ref[...])
    h = jnp.dot(h, wc_ref[...], preferred_element_type=jnp.float32)
    h = jax.nn.relu(h + bc_ref[...])
    o_ref[...] = jnp.concatenate(
        [h, jnp.zeros((h.shape[0], 64), jnp.float32)], axis=1)


def _t2(hin, wa, ba, wb, bb, wc, bc):
    TM = 1024
    return pl.pallas_call(
        _t2_body,
        grid=(B * N // TM,),
        in_specs=[
            pl.BlockSpec((TM, 16), lambda t: (t, 0)),
            pl.BlockSpec((16, 64), lambda t: (0, 0)),
            pl.BlockSpec((1, 64), lambda t: (0, 0)),
            pl.BlockSpec((64, 64), lambda t: (0, 0)),
            pl.BlockSpec((1, 64), lambda t: (0, 0)),
            pl.BlockSpec((64, 64), lambda t: (0, 0)),
            pl.BlockSpec((1, 64), lambda t: (0, 0)),
        ],
        out_specs=pl.BlockSpec((TM, 128), lambda t: (t, 0)),
        out_shape=jax.ShapeDtypeStruct((B * N, 128), jnp.float32),
    )(hin, wa, ba, wb, bb, wc, bc)


def _t3_body(x_ref, a_ref, ws_ref, wn_ref, bb_ref, o_ref):
    y = jnp.dot(x_ref[...], ws_ref[...], preferred_element_type=jnp.float32)
    y = y + jnp.dot(a_ref[...], wn_ref[...], preferred_element_type=jnp.float32)
    o_ref[...] = jax.nn.relu(y + bb_ref[...])


def _t3(x1, a1, ws, wn, bb, Cin, Cout):
    TM = 1024
    return pl.pallas_call(
        _t3_body,
        grid=(B * N // TM,),
        in_specs=[
            pl.BlockSpec((TM, Cin), lambda t: (t, 0)),
            pl.BlockSpec((TM, Cin), lambda t: (t, 0)),
            pl.BlockSpec((Cin, Cout), lambda t: (0, 0)),
            pl.BlockSpec((Cin, Cout), lambda t: (0, 0)),
            pl.BlockSpec((1, Cout), lambda t: (0, 0)),
        ],
        out_specs=pl.BlockSpec((TM, Cout), lambda t: (t, 0)),
        out_shape=jax.ShapeDtypeStruct((B * N, Cout), jnp.float32),
    )(x1, a1, ws, wn, bb)


def _t4_body(x_ref, a_ref, mp_ref, ws_ref, wn_ref, bb_ref, o_ref):
    y = jnp.dot(x_ref[...], ws_ref[...], preferred_element_type=jnp.float32)
    y = y + jnp.dot(a_ref[...], wn_ref[...], preferred_element_type=jnp.float32)
    y = jax.nn.relu(y + bb_ref[...]) + mp_ref[...]
    o_ref[...] = jnp.max(y, axis=0, keepdims=True)[None]


def _t4(x2, a2, mpen, ws, wn, bb):
    TM = 1024
    return pl.pallas_call(
        _t4_body,
        grid=(B * N // TM,),
        in_specs=[
            pl.BlockSpec((TM, 128), lambda t: (t, 0)),
            pl.BlockSpec((TM, 128), lambda t: (t, 0)),
            pl.BlockSpec((TM, 1), lambda t: (t, 0)),
            pl.BlockSpec((128, 2 * NF), lambda t: (0, 0)),
            pl.BlockSpec((128, 2 * NF), lambda t: (0, 0)),
            pl.BlockSpec((1, 2 * NF), lambda t: (0, 0)),
        ],
        out_specs=pl.BlockSpec(
            (1, 1, 2 * NF), lambda t: ((t % 2) * B + t // 2, 0, 0)),
        out_shape=jax.ShapeDtypeStruct((2 * B, 1, 2 * NF), jnp.float32),
    )(x2, a2, mpen, ws, wn, bb)


def _t5_body(p_ref, wa_ref, ba_ref, wb_ref, bb_ref, o_ref):
    p = jnp.maximum(p_ref[0], p_ref[1])
    z = jnp.dot(p, wa_ref[...], preferred_element_type=jnp.float32)
    z = jax.nn.relu(z + ba_ref[...])
    z = jnp.dot(z, wb_ref[...], preferred_element_type=jnp.float32)
    o_ref[...] = jax.nn.relu(z + bb_ref[...])


def _t5(part, wa, ba, wb, bb):
    return pl.pallas_call(
        _t5_body,
        out_shape=jax.ShapeDtypeStruct((B, NF), jnp.float32),
    )(part, wa, ba, wb, bb)


def kernel(x, mask, w1a, b1a, g1a, be1a, w1b, b1b, g1b, be1b, w1c, b1c, g1c, be1c, gs1, gn1, gb1, gs2, gn2, gb2, w2a, b2a, g2a, be2a, w2b, b2b, g2b, be2b):
    pts = jnp.transpose(x, (0, 2, 1))
    ptsA = jnp.concatenate(
        [-2.0 * pts, jnp.ones((B, N, 1), jnp.float32),
         jnp.zeros((B, N, 4), jnp.float32)], axis=-1)
    mask3 = mask.reshape(B, 1, N)

    spen, gmin = _t1(ptsA, x, mask3)
    spen2 = spen.reshape(B, N)
    gminf = gmin.reshape(B * N * 128)

    glf, hinf = _sc1(x.reshape(B * 3 * N), spen2.reshape(B * N), gminf)
    hin = hinf.reshape(B * N, 16)

    s1a = _scale(g1a)
    wa = jnp.zeros((16, 64), jnp.float32).at[:12].set(w1a.T) * s1a[None, :]
    ba = (b1a * s1a + be1a)[None, :]
    s1b = _scale(g1b)
    wb = w1b.T * s1b[None, :]
    bb_ = (b1b * s1b + be1b)[None, :]
    s1c = _scale(g1c)
    wc = w1c.T * s1c[None, :]
    bc = (b1c * s1c + be1c)[None, :]

    x1 = _t2(hin, wa, ba, wb, bb_, wc, bc)
    a1 = _make_gathermax(128)(x1, glf).reshape(B * N, 128)
    ws1 = jnp.zeros((128, 128), jnp.float32).at[:64].set(gs1.T)
    wn1 = jnp.zeros((128, 128), jnp.float32).at[:64].set(gn1.T)
    x2 = _t3(x1, a1, ws1, wn1, gb1[None, :], 128, 128)
    a2 = _make_gathermax(128)(x2, glf).reshape(B * N, 128)

    mpen = ((1 - mask).astype(jnp.float32) * NEG).reshape(B * N, 1)
    part = _t4(x2, a2, mpen, gs2.T, gn2.T, gb2[None, :]).reshape(2, B, 2 * NF)

    s2a = _scale(g2a)
    w2ae = w2a.T * s2a[None, :]
    b2ae = (b2a * s2a + be2a)[None, :]
    s2b = _scale(g2b)
    w2be = w2b.T * s2b[None, :]
    b2be = (b2b * s2b + be2b)[None, :]
    return _t5(part, w2ae, b2ae, w2be, b2be)


# speedup vs baseline: 10.0928x; 1.2417x over previous
import functools

import jax
import jax.numpy as jnp
from jax import lax
from jax.experimental import pallas as pl
from jax.experimental.pallas import tpu as pltpu
from jax.experimental.pallas import tpu_sc as plsc

B = 8
N = 2048
K = 16
NF = 512
BIG = 1e30
NEG = -1e37

_NC = 2
_NS = 16
NW = _NC * _NS
RPW = (B * N) // NW
RT = 256

_mesh = plsc.VectorSubcoreMesh(
    core_axis_name="c", subcore_axis_name="s", num_cores=_NC, num_subcores=_NS)


def _scale(g):
    return g / jnp.sqrt(1.0 + 1e-5)


def _t1_body(ptsA_ref, x_ref, m_ref, spen_ref, gmin_ref):
    xb = x_ref[0]
    m = m_ref[0].astype(jnp.float32)
    pen = (1.0 - m) * BIG
    x0 = xb[0:1]
    x1 = xb[1:2]
    x2 = xb[2:3]
    spen = x0 * x0 + x1 * x1 + x2 * x2 + pen
    spen_ref[0] = spen
    kb = jnp.concatenate([xb, spen, jnp.zeros((4, N), jnp.float32)], axis=0)
    a = ptsA_ref[0]
    d = lax.dot_general(a, kb, (((1,), (0,)), ((), ())),
                        preferred_element_type=jnp.float32)
    gm = d[:, 0:128]
    for i in range(1, 16):
        gm = jnp.minimum(gm, d[:, i * 128:(i + 1) * 128])
    gmin_ref[0] = gm


def _t1(ptsA, x, mask3):
    return pl.pallas_call(
        _t1_body,
        grid=(B, N // RT),
        in_specs=[
            pl.BlockSpec((1, RT, 8), lambda b, r: (b, r, 0)),
            pl.BlockSpec((1, 3, N), lambda b, r: (b, 0, 0)),
            pl.BlockSpec((1, 1, N), lambda b, r: (b, 0, 0)),
        ],
        out_specs=[
            pl.BlockSpec((1, 1, N), lambda b, r: (b, 0, 0)),
            pl.BlockSpec((1, RT, 128), lambda b, r: (b, r, 0)),
        ],
        out_shape=[
            jax.ShapeDtypeStruct((B, 1, N), jnp.float32),
            jax.ShapeDtypeStruct((B, N, 128), jnp.float32),
        ],
    )(ptsA, x, mask3)


def _vgather(vec, idx):
    dnums = lax.GatherDimensionNumbers(
        offset_dims=(), collapsed_slice_dims=(0,), start_index_map=(0,))
    return lax.gather(vec, idx[:, None], dnums, slice_sizes=(1,),
                      mode=lax.GatherScatterMode.PROMISE_IN_BOUNDS)


def _skv(k, v):
    sk, sv = lax.sort((k, v), dimension=0, num_keys=1)
    return sk, sv


def _merge16(ak, av, bk, bv):
    rk = lax.rev(bk, (0,))
    rv = lax.rev(bv, (0,))
    sel = ak <= rk
    mk = jnp.where(sel, ak, rk)
    mv = jnp.where(sel, av, rv)
    return _skv(mk, mv)


def _sc1_body(xh, spen, gminf, glo, hino,
              xt, yt, zt, st, gb0, gb1, glbuf, hbuf, sem0, sem1):
    wid = lax.axis_index("c") * _NS + lax.axis_index("s")
    b = wid // 4
    q = wid % 4
    base = b * N

    pltpu.sync_copy(xh.at[pl.ds((b * 3 + 0) * N, N)], xt)
    pltpu.sync_copy(xh.at[pl.ds((b * 3 + 1) * N, N)], yt)
    pltpu.sync_copy(xh.at[pl.ds((b * 3 + 2) * N, N)], zt)
    pltpu.sync_copy(spen.at[pl.ds(b * N, N)], st)

    ji = lax.iota(jnp.int32, 16)
    j15 = jnp.full((16,), 15, jnp.int32)

    def gsrc(ci):
        return gminf.at[pl.ds((base + q * RPW + ci * 64) * 128, 64 * 128)]

    def phase_a(gbuf, i2r):
        pairs = []
        for j in range(8):
            gj = gbuf[pl.ds(i2r * 128 + j * 16, 16)]
            pairs.append(_skv(gj, ji + (16 * j)))
        while len(pairs) > 1:
            pairs = [_merge16(*pairs[k], *pairs[k + 1])
                     for k in range(0, len(pairs), 2)]
        return pairs[0]

    def do_rows(gbuf, c):
        def row_pair(i2, carry2):
            rows = (2 * i2, 2 * i2 + 1)
            state = []
            for r in rows:
                gk, gv = phase_a(gbuf, r)
                nv = jnp.full((16,), q * RPW + c * 64 + r, jnp.int32)
                xr = plsc.load_gather(xt, [nv])
                yr = plsc.load_gather(yt, [nv])
                zr = plsc.load_gather(zt, [nv])
                state.append((gk, gv, xr, yr, zr))

            def bstep(j, carry3):
                jv = jnp.full((16,), j, jnp.int32)
                outs = []
                for t in range(2):
                    rk0, rv0 = carry3[t]
                    gk, gv, xr, yr, zr = state[t]
                    gkj = _vgather(gk, jv)
                    gvj = _vgather(gv, jv)
                    r15 = _vgather(rk0, j15)
                    go = jnp.max((r15 > gkj).astype(jnp.int32)) > 0

                    def do(_, rk0=rk0, rv0=rv0, gvj=gvj,
                           xr=xr, yr=yr, zr=zr):
                        cols = gvj + ji * 128
                        xc = plsc.load_gather(xt, [cols])
                        yc = plsc.load_gather(yt, [cols])
                        zc = plsc.load_gather(zt, [cols])
                        sc = plsc.load_gather(st, [cols])
                        d = sc - 2.0 * (xc * xr + yc * yr + zc * zr)
                        ck, cv = _skv(d, cols)
                        return _merge16(rk0, rv0, ck, cv)

                    outs.append(
                        lax.cond(go, do,
                                 lambda _, rk0=rk0, rv0=rv0: (rk0, rv0), None))
                return tuple(outs)

            init1 = (jnp.full((16,), 3.0e38, jnp.float32),
                     jnp.zeros((16,), jnp.int32))
            res = lax.fori_loop(0, 16, bstep, (init1, init1))

            for t, r in enumerate(rows):
                rk, rv = res[t]
                _, _, xr, yr, zr = state[t]
                row = c * 64 + r
                glbuf[pl.ds(row * K, K)] = rv + base
                xn = plsc.load_gather(xt, [rv])
                yn = plsc.load_gather(yt, [rv])
                zn = plsc.load_gather(zt, [rv])
                sx = jnp.sum(xn)
                sy = jnp.sum(yn)
                sz = jnp.sum(zn)
                cxx = jnp.sum(xn * xn) - sx * sx * (1.0 / K)
                cxy = jnp.sum(xn * yn) - sx * sy * (1.0 / K)
                cxz = jnp.sum(xn * zn) - sx * sz * (1.0 / K)
                cyy = jnp.sum(yn * yn) - sy * sy * (1.0 / K)
                cyz = jnp.sum(yn * zn) - sy * sz * (1.0 / K)
                czz = jnp.sum(zn * zn) - sz * sz * (1.0 / K)
                hrow = jnp.zeros((16,), jnp.float32)
                hrow = jnp.where(ji == 0, xr, hrow)
                hrow = jnp.where(ji == 1, yr, hrow)
                hrow = jnp.where(ji == 2, zr, hrow)
                for lane, val in ((3, cxx), (4, cxy), (5, cxz), (6, cxy),
                                  (7, cyy), (8, cyz), (9, cxz), (10, cyz),
                                  (11, czz)):
                    hrow = jnp.where(ji == lane, jnp.full((16,), val), hrow)
                hbuf[pl.ds(row * 16, 16)] = hrow
            return carry2

        lax.fori_loop(0, 32, row_pair, 0)

    pltpu.make_async_copy(gsrc(0), gb0, sem0).start()

    def pair_body(cp, carry):
        c0 = 2 * cp
        pltpu.make_async_copy(gsrc(c0), gb0, sem0).wait()
        pltpu.make_async_copy(gsrc(c0 + 1), gb1, sem1).start()
        do_rows(gb0, c0)

        @pl.when(cp < 3)
        def _():
            pltpu.make_async_copy(gsrc(c0 + 2), gb0, sem0).start()

        pltpu.make_async_copy(gsrc(c0 + 1), gb1, sem1).wait()
        do_rows(gb1, c0 + 1)
        return carry

    lax.fori_loop(0, 4, pair_body, 0)

    pltpu.sync_copy(glbuf, glo.at[pl.ds(wid * RPW * K, RPW * K)])
    pltpu.sync_copy(hbuf, hino.at[pl.ds(wid * RPW * 16, RPW * 16)])


def _sc1(x, spen2, gminf):
    fn = pl.kernel(
        _sc1_body,
        compiler_params=pltpu.CompilerParams(needs_layout_passes=False),
        out_type=(
            jax.ShapeDtypeStruct((B * N * K,), jnp.int32),
            jax.ShapeDtypeStruct((B * N * 16,), jnp.float32),
        ),
        mesh=_mesh,
        scratch_types=[
            pltpu.VMEM((N,), jnp.float32),
            pltpu.VMEM((N,), jnp.float32),
            pltpu.VMEM((N,), jnp.float32),
            pltpu.VMEM((N,), jnp.float32),
            pltpu.VMEM((64 * 128,), jnp.float32),
            pltpu.VMEM((64 * 128,), jnp.float32),
            pltpu.VMEM((RPW * K,), jnp.int32),
            pltpu.VMEM((RPW * 16,), jnp.float32),
            pltpu.SemaphoreType.DMA,
            pltpu.SemaphoreType.DMA,
        ],
    )
    return fn(x, spen2, gminf)


def _make_gathermax(C):
    CH = 8

    def body(feat, glf, outf, idxb, rb0, rb1, outb, sem0, sem1):
        wid = lax.axis_index("c") * _NS + lax.axis_index("s")
        pltpu.sync_copy(glf.at[pl.ds(wid * RPW * K, RPW * K)], idxb)

        def isrc(ch):
            return feat.at[idxb.at[pl.ds(ch * CH * K, CH * K)]]

        def compute(rowsb, ch):
            for p in range(CH):
                for cc in range(C // 16):
                    acc = rowsb[p * K + 0, pl.ds(cc * 16, 16)]
                    for k in range(1, K):
                        acc = jnp.maximum(
                            acc, rowsb[p * K + k, pl.ds(cc * 16, 16)])
                    outb[pl.ds((ch * CH + p) * C + cc * 16, 16)] = acc

        nch = RPW // CH
        pltpu.make_async_copy(isrc(0), rb0, sem0).start()

        def pair_body(cp, carry):
            c0 = 2 * cp
            pltpu.make_async_copy(isrc(c0), rb0, sem0).wait()
            pltpu.make_async_copy(isrc(c0 + 1), rb1, sem1).start()
            compute(rb0, c0)

            @pl.when(cp < nch // 2 - 1)
            def _():
                pltpu.make_async_copy(isrc(c0 + 2), rb0, sem0).start()

            pltpu.make_async_copy(isrc(c0 + 1), rb1, sem1).wait()
            compute(rb1, c0 + 1)
            return carry

        lax.fori_loop(0, nch // 2, pair_body, 0)
        pltpu.sync_copy(outb, outf.at[pl.ds(wid * RPW * C, RPW * C)])

    fn = pl.kernel(
        body,
        compiler_params=pltpu.CompilerParams(needs_layout_passes=False),
        out_type=jax.ShapeDtypeStruct((B * N * C,), jnp.float32),
        mesh=_mesh,
        scratch_types=[
            pltpu.VMEM((RPW * K,), jnp.int32),
            pltpu.VMEM((CH * K, C), jnp.float32),
            pltpu.VMEM((CH * K, C), jnp.float32),
            pltpu.VMEM((RPW * C,), jnp.float32),
            pltpu.SemaphoreType.DMA,
            pltpu.SemaphoreType.DMA,
        ],
    )
    return fn


def _t2_body(h_ref, wa_ref, ba_ref, wb_ref, bb_ref, wc_ref, bc_ref, o_ref):
    h = jnp.dot(h_ref[...], wa_ref[...], preferred_element_type=jnp.float32)
    h = jax.nn.relu(h + ba_ref[...])
    h = jnp.dot(h, wb_ref[...], preferred_element_type=jnp.float32)
    h = jax.nn.relu(h + bb_ref[...])
    h = jnp.dot(h, wc_ref[...], preferred_element_type=jnp.float32)
    h = jax.nn.relu(h + bc_ref[...])
    o_ref[...] = jnp.concatenate(
        [h, jnp.zeros((h.shape[0], 64), jnp.float32)], axis=1)


def _t2(hin, wa, ba, wb, bb, wc, bc):
    TM = 1024
    return pl.pallas_call(
        _t2_body,
        grid=(B * N // TM,),
        in_specs=[
            pl.BlockSpec((TM, 16), lambda t: (t, 0)),
            pl.BlockSpec((16, 64), lambda t: (0, 0)),
            pl.BlockSpec((1, 64), lambda t: (0, 0)),
            pl.BlockSpec((64, 64), lambda t: (0, 0)),
            pl.BlockSpec((1, 64), lambda t: (0, 0)),
            pl.BlockSpec((64, 64), lambda t: (0, 0)),
            pl.BlockSpec((1, 64), lambda t: (0, 0)),
        ],
        out_specs=pl.BlockSpec((TM, 128), lambda t: (t, 0)),
        out_shape=jax.ShapeDtypeStruct((B * N, 128), jnp.float32),
    )(hin, wa, ba, wb, bb, wc, bc)


def _t3_body(x_ref, a_ref, ws_ref, wn_ref, bb_ref, o_ref):
    y = jnp.dot(x_ref[...], ws_ref[...], preferred_element_type=jnp.float32)
    y = y + jnp.dot(a_ref[...], wn_ref[...], preferred_element_type=jnp.float32)
    o_ref[...] = jax.nn.relu(y + bb_ref[...])


def _t3(x1, a1, ws, wn, bb, Cin, Cout):
    TM = 1024
    return pl.pallas_call(
        _t3_body,
        grid=(B * N // TM,),
        in_specs=[
            pl.BlockSpec((TM, Cin), lambda t: (t, 0)),
            pl.BlockSpec((TM, Cin), lambda t: (t, 0)),
            pl.BlockSpec((Cin, Cout), lambda t: (0, 0)),
            pl.BlockSpec((Cin, Cout), lambda t: (0, 0)),
            pl.BlockSpec((1, Cout), lambda t: (0, 0)),
        ],
        out_specs=pl.BlockSpec((TM, Cout), lambda t: (t, 0)),
        out_shape=jax.ShapeDtypeStruct((B * N, Cout), jnp.float32),
    )(x1, a1, ws, wn, bb)


def _t4_body(x_ref, a_ref, mp_ref, ws_ref, wn_ref, bb_ref, o_ref):
    y = jnp.dot(x_ref[...], ws_ref[...], preferred_element_type=jnp.float32)
    y = y + jnp.dot(a_ref[...], wn_ref[...], preferred_element_type=jnp.float32)
    y = jax.nn.relu(y + bb_ref[...]) + mp_ref[...]
    o_ref[...] = jnp.max(y, axis=0, keepdims=True)[None]


def _t4(x2, a2, mpen, ws, wn, bb):
    TM = 1024
    return pl.pallas_call(
        _t4_body,
        grid=(B * N // TM,),
        in_specs=[
            pl.BlockSpec((TM, 128), lambda t: (t, 0)),
            pl.BlockSpec((TM, 128), lambda t: (t, 0)),
            pl.BlockSpec((TM, 1), lambda t: (t, 0)),
            pl.BlockSpec((128, 2 * NF), lambda t: (0, 0)),
            pl.BlockSpec((128, 2 * NF), lambda t: (0, 0)),
            pl.BlockSpec((1, 2 * NF), lambda t: (0, 0)),
        ],
        out_specs=pl.BlockSpec(
            (1, 1, 2 * NF), lambda t: ((t % 2) * B + t // 2, 0, 0)),
        out_shape=jax.ShapeDtypeStruct((2 * B, 1, 2 * NF), jnp.float32),
    )(x2, a2, mpen, ws, wn, bb)


def _t5_body(p_ref, wa_ref, ba_ref, wb_ref, bb_ref, o_ref):
    p = jnp.maximum(p_ref[0], p_ref[1])
    z = jnp.dot(p, wa_ref[...], preferred_element_type=jnp.float32)
    z = jax.nn.relu(z + ba_ref[...])
    z = jnp.dot(z, wb_ref[...], preferred_element_type=jnp.float32)
    o_ref[...] = jax.nn.relu(z + bb_ref[...])


def _t5(part, wa, ba, wb, bb):
    return pl.pallas_call(
        _t5_body,
        out_shape=jax.ShapeDtypeStruct((B, NF), jnp.float32),
    )(part, wa, ba, wb, bb)


def kernel(x, mask, w1a, b1a, g1a, be1a, w1b, b1b, g1b, be1b, w1c, b1c, g1c, be1c, gs1, gn1, gb1, gs2, gn2, gb2, w2a, b2a, g2a, be2a, w2b, b2b, g2b, be2b):
    pts = jnp.transpose(x, (0, 2, 1))
    ptsA = jnp.concatenate(
        [-2.0 * pts, jnp.ones((B, N, 1), jnp.float32),
         jnp.zeros((B, N, 4), jnp.float32)], axis=-1)
    mask3 = mask.reshape(B, 1, N)

    spen, gmin = _t1(ptsA, x, mask3)
    spen2 = spen.reshape(B, N)
    gminf = gmin.reshape(B * N * 128)

    glf, hinf = _sc1(x.reshape(B * 3 * N), spen2.reshape(B * N), gminf)
    hin = hinf.reshape(B * N, 16)

    s1a = _scale(g1a)
    wa = jnp.zeros((16, 64), jnp.float32).at[:12].set(w1a.T) * s1a[None, :]
    ba = (b1a * s1a + be1a)[None, :]
    s1b = _scale(g1b)
    wb = w1b.T * s1b[None, :]
    bb_ = (b1b * s1b + be1b)[None, :]
    s1c = _scale(g1c)
    wc = w1c.T * s1c[None, :]
    bc = (b1c * s1c + be1c)[None, :]

    x1 = _t2(hin, wa, ba, wb, bb_, wc, bc)
    a1 = _make_gathermax(128)(x1, glf).reshape(B * N, 128)
    ws1 = jnp.zeros((128, 128), jnp.float32).at[:64].set(gs1.T)
    wn1 = jnp.zeros((128, 128), jnp.float32).at[:64].set(gn1.T)
    x2 = _t3(x1, a1, ws1, wn1, gb1[None, :], 128, 128)
    a2 = _make_gathermax(128)(x2, glf).reshape(B * N, 128)

    mpen = ((1 - mask).astype(jnp.float32) * NEG).reshape(B * N, 1)
    part = _t4(x2, a2, mpen, gs2.T, gn2.T, gb2[None, :]).reshape(2, B, 2 * NF)

    s2a = _scale(g2a)
    w2ae = w2a.T * s2a[None, :]
    b2ae = (b2a * s2a + be2a)[None, :]
    s2b = _scale(g2b)
    w2be = w2b.T * s2b[None, :]
    b2be = (b2b * s2b + be2b)[None, :]
    return _t5(part, w2ae, b2ae, w2be, b2be)
